# baseline (speedup 1.0000x reference)
import functools

import jax
import jax.numpy as jnp
from jax import lax
from jax.experimental import pallas as pl
from jax.experimental.pallas import tpu as pltpu
from jax.experimental.pallas import tpu_sc as plsc

C = 128
K = 16
NUM_CORES = 2
NUM_SUBCORES = 16
NUM_WORKERS = NUM_CORES * NUM_SUBCORES


def _ord_u32(b):
    neg = b >= jnp.uint32(0x80000000)
    return jnp.where(neg, ~b, b | jnp.uint32(0x80000000))


def _unord_u32(o):
    neg = o < jnp.uint32(0x80000000)
    return jnp.where(neg, ~o, o - jnp.uint32(0x80000000))


def _bf16_step(bits, delta):
    return _unord_u32(_ord_u32(bits) + jnp.uint32(delta * 0x10000 & 0xFFFFFFFF))


def _adjust_thresholds(t):
    tf = t.astype(jnp.float32)
    u = lax.bitcast_convert_type(tf, jnp.uint32)
    r = u + jnp.uint32(0x7FFF) + ((u >> jnp.uint32(16)) & jnp.uint32(1))
    rb = r & jnp.uint32(0xFFFF0000)
    rf = lax.bitcast_convert_type(rb, jnp.float32)
    down = jnp.where(rf > tf, _bf16_step(rb, -1), rb)
    up = _bf16_step(down, 1)
    m = 0.5 * (lax.bitcast_convert_type(down, jnp.float32)
               + lax.bitcast_convert_type(up, jnp.float32))
    mb = lax.bitcast_convert_type(m, jnp.uint32)
    m_prev = lax.bitcast_convert_type(
        _unord_u32(_ord_u32(mb) - jnp.uint32(1)), jnp.float32)
    even = ((up >> jnp.uint32(16)) & jnp.uint32(1)) == 0
    return jnp.where(even, m_prev, m)


_CHUNK = 8


def _sc_encode_body_nc(nc, n_tokens, in_features, tok_off, x_hbm, dims_hbm,
                       thr_hbm, idx_hbm, dims_v, thr_v, row_v, ch_v, idx_v,
                       in_sem):
    tpw = n_tokens // (nc * NUM_SUBCORES)
    wid = lax.axis_index("s") * nc + lax.axis_index("c")
    tok0 = tok_off + wid * tpw

    pltpu.sync_copy(dims_hbm, dims_v)
    pltpu.sync_copy(thr_hbm, thr_v)

    nchunks = tpw // _CHUNK

    def chunk_copy(ci):
        slot = lax.rem(ci, 2) * _CHUNK
        return pltpu.make_async_copy(
            x_hbm.at[pl.ds(tok0 + ci * _CHUNK, _CHUNK)],
            row_v.at[pl.ds(slot, _CHUNK)], in_sem)

    chunk_copy(0).start()

    def p1_chunk(ci, carry):
        @pl.when(ci + 1 < nchunks)
        def _():
            chunk_copy(ci + 1).start()
        chunk_copy(ci).wait()
        rowbase = lax.rem(ci, 2) * _CHUNK

        @plsc.parallel_loop(0, _CHUNK, unroll=2)
        def p1_tok(r):
            rv = jnp.full((16,), rowbase + r, jnp.int32)
            cb = (ci * _CHUNK + r) * (4 * C)
            for j in range(4 * C // 16):
                ivec = dims_v[pl.ds(j * 16, 16)]
                ch_v[pl.ds(cb + j * 16, 16)] = plsc.load_gather(
                    row_v, [rv, ivec])

        return carry

    lax.fori_loop(0, nchunks, p1_chunk, 0)

    iota = lax.iota(jnp.int32, 16)
    ones16 = jnp.ones((16,), jnp.int32)
    zero16 = jnp.zeros((16,), jnp.int32)

    for g in range(C // 16):
        t0 = thr_v[pl.ds(g * 16, 16)]
        t1a = thr_v[pl.ds(C + g * 16, 16)]
        t1b = thr_v[pl.ds(2 * C + g * 16, 16)]
        t2a = thr_v[pl.ds(3 * C + g * 16, 16)]
        t2b = thr_v[pl.ds(4 * C + g * 16, 16)]
        t2c = thr_v[pl.ds(5 * C + g * 16, 16)]
        t2d = thr_v[pl.ds(6 * C + g * 16, 16)]
        g3base = iota + (7 * C + g * 16)

        def one_token(t):
            cb = t * (4 * C) + g * 16
            v0 = ch_v[pl.ds(cb, 16)]
            v1 = ch_v[pl.ds(cb + C, 16)]
            v2 = ch_v[pl.ds(cb + 2 * C, 16)]
            v3 = ch_v[pl.ds(cb + 3 * C, 16)]
            b0 = v0 > t0
            b1 = v1 > jnp.where(b0, t1b, t1a)
            td2 = jnp.where(b0, jnp.where(b1, t2d, t2c),
                            jnp.where(b1, t2b, t2a))
            b2 = v2 > td2
            off = (jnp.where(b0, 4, zero16) + jnp.where(b1, 2, zero16)
                   + jnp.where(b2, ones16, zero16))
            t3 = plsc.load_gather(thr_v, [g3base + off * C])
            leaf = 2 * off + jnp.where(v3 > t3, ones16, zero16)
            idx_v[pl.ds(t * C + g * 16, 16)] = leaf.astype(jnp.float32)

        plsc.parallel_loop(0, tpw, unroll=8)(one_token)

    pltpu.sync_copy(idx_v, idx_hbm.at[pl.ds(wid * tpw * C, tpw * C)])


def _sc_encode(x, dims_r, thr_r, tok_off=0, n_tokens=None):
    total_tokens, in_features = x.shape
    if n_tokens is None:
        n_tokens = total_tokens
    tpw = n_tokens // NUM_WORKERS
    mesh = plsc.VectorSubcoreMesh(core_axis_name="c", subcore_axis_name="s",
                                  num_cores=NUM_CORES,
                                  num_subcores=NUM_SUBCORES)
    body = functools.partial(_sc_encode_body_nc, NUM_CORES, n_tokens,
                             in_features, tok_off)
    fn = pl.kernel(
        body,
        out_type=jax.ShapeDtypeStruct((n_tokens * C,), jnp.float32),
        mesh=mesh,
        scratch_types=[
            pltpu.VMEM((4 * C,), jnp.int32),
            pltpu.VMEM((15 * C,), jnp.float32),
            pltpu.VMEM((2 * _CHUNK, in_features), jnp.float32),
            pltpu.VMEM((tpw * 4 * C,), jnp.float32),
            pltpu.VMEM((tpw * C,), jnp.float32),
            pltpu.SemaphoreType.DMA,
        ],
        compiler_params=pltpu.CompilerParams(needs_layout_passes=False),
    )
    return fn(x, dims_r, thr_r)


def _tc_decode_body(idx_ref, r_ref, l_ref, o_ref):
    bm = idx_ref.shape[0]
    idxb = idx_ref[...].astype(jnp.bfloat16)
    rep = lax.dot_general(idxb, r_ref[...], (((1,), (0,)), ((), ())),
                          preferred_element_type=jnp.float32)
    kv = lax.broadcasted_iota(jnp.int32, (bm, C * K), 1) & (K - 1)
    e = (rep == kv.astype(jnp.float32)).astype(jnp.bfloat16)
    o_ref[...] = lax.dot_general(e, l_ref[...], (((1,), (1,)), ((), ())),
                                 preferred_element_type=jnp.float32)


def _tc_decode(idxm, rep_mat, lut_flat):
    n_tokens = idxm.shape[0]
    out_features = lut_flat.shape[0]
    bm = 512
    grid = (n_tokens // bm,)
    return pl.pallas_call(
        _tc_decode_body,
        grid=grid,
        in_specs=[
            pl.BlockSpec((bm, C), lambda i: (i, 0)),
            pl.BlockSpec((C, C * K), lambda i: (0, 0)),
            pl.BlockSpec((out_features, C * K), lambda i: (0, 0)),
        ],
        out_specs=pl.BlockSpec((bm, out_features), lambda i: (i, 0)),
        out_shape=jax.ShapeDtypeStruct((n_tokens, out_features), jnp.float32),
    )(idxm, rep_mat, lut_flat)


def kernel(inputMatrix, dims, thresholds, lut, selectionMatrix, treeDesMat):
    n_tokens, in_features = inputMatrix.shape
    out_features = lut.shape[0]

    dims_r = dims.reshape(C, 4).T.reshape(-1)
    thr_r = _adjust_thresholds(thresholds.reshape(-1)).reshape(C, 15).T.reshape(-1)

    rep_mat = (lax.broadcasted_iota(jnp.int32, (C, C * K), 1) // K ==
               lax.broadcasted_iota(jnp.int32, (C, C * K), 0)
               ).astype(jnp.bfloat16)
    lut_flat = lut.reshape(out_features, C * K).astype(jnp.bfloat16)

    half = n_tokens // 2
    idx1 = _sc_encode(inputMatrix, dims_r, thr_r, 0, half)
    idx2 = _sc_encode(inputMatrix, dims_r, thr_r, half, half)
    out1 = _tc_decode(idx1.reshape(half, C), rep_mat, lut_flat)
    out2 = _tc_decode(idx2.reshape(half, C), rep_mat, lut_flat)
    return jnp.concatenate([out1, out2], axis=0)


# speedup vs baseline: 1.2881x; 1.2881x over previous
import functools

import jax
import jax.numpy as jnp
from jax import lax
from jax.experimental import pallas as pl
from jax.experimental.pallas import tpu as pltpu
from jax.experimental.pallas import tpu_sc as plsc

C = 128
K = 16
NUM_CORES = 2
NUM_SUBCORES = 16
NUM_WORKERS = NUM_CORES * NUM_SUBCORES


def _ord_u32(b):
    neg = b >= jnp.uint32(0x80000000)
    return jnp.where(neg, ~b, b | jnp.uint32(0x80000000))


def _unord_u32(o):
    neg = o < jnp.uint32(0x80000000)
    return jnp.where(neg, ~o, o - jnp.uint32(0x80000000))


def _bf16_step(bits, delta):
    return _unord_u32(_ord_u32(bits) + jnp.uint32(delta * 0x10000 & 0xFFFFFFFF))


def _adjust_thresholds(t):
    tf = t.astype(jnp.float32)
    u = lax.bitcast_convert_type(tf, jnp.uint32)
    r = u + jnp.uint32(0x7FFF) + ((u >> jnp.uint32(16)) & jnp.uint32(1))
    rb = r & jnp.uint32(0xFFFF0000)
    rf = lax.bitcast_convert_type(rb, jnp.float32)
    down = jnp.where(rf > tf, _bf16_step(rb, -1), rb)
    up = _bf16_step(down, 1)
    m = 0.5 * (lax.bitcast_convert_type(down, jnp.float32)
               + lax.bitcast_convert_type(up, jnp.float32))
    mb = lax.bitcast_convert_type(m, jnp.uint32)
    m_prev = lax.bitcast_convert_type(
        _unord_u32(_ord_u32(mb) - jnp.uint32(1)), jnp.float32)
    even = ((up >> jnp.uint32(16)) & jnp.uint32(1)) == 0
    return jnp.where(even, m_prev, m)


_CHUNK = 8


def _sc_encode_body_nc(nc, n_tokens, in_features, tok_off, x_hbm, dims_hbm,
                       thr_hbm, idx_hbm, dims_v, thr_v, row_v, ch_v, idx_v,
                       in_sem):
    tpw = n_tokens // (nc * NUM_SUBCORES)
    wid = lax.axis_index("s") * nc + lax.axis_index("c")
    tok0 = tok_off + wid * tpw

    pltpu.sync_copy(dims_hbm, dims_v)
    pltpu.sync_copy(thr_hbm, thr_v)

    nchunks = tpw // _CHUNK

    def chunk_copy(ci):
        slot = lax.rem(ci, 2) * _CHUNK
        return pltpu.make_async_copy(
            x_hbm.at[pl.ds(tok0 + ci * _CHUNK, _CHUNK)],
            row_v.at[pl.ds(slot, _CHUNK)], in_sem)

    chunk_copy(0).start()

    def p1_chunk(ci, carry):
        @pl.when(ci + 1 < nchunks)
        def _():
            chunk_copy(ci + 1).start()
        chunk_copy(ci).wait()
        rowbase = lax.rem(ci, 2) * _CHUNK

        @plsc.parallel_loop(0, _CHUNK, unroll=2)
        def p1_tok(r):
            rv = jnp.full((16,), rowbase + r, jnp.int32)
            cb = (ci * _CHUNK + r) * (4 * C)
            for j in range(4 * C // 16):
                ivec = dims_v[pl.ds(j * 16, 16)]
                ch_v[pl.ds(cb + j * 16, 16)] = plsc.load_gather(
                    row_v, [rv, ivec])

        return carry

    lax.fori_loop(0, nchunks, p1_chunk, 0)

    iota = lax.iota(jnp.int32, 16)
    ones16 = jnp.ones((16,), jnp.int32)
    zero16 = jnp.zeros((16,), jnp.int32)

    for g in range(C // 16):
        t0 = thr_v[pl.ds(g * 16, 16)]
        t1a = thr_v[pl.ds(C + g * 16, 16)]
        t1b = thr_v[pl.ds(2 * C + g * 16, 16)]
        t2a = thr_v[pl.ds(3 * C + g * 16, 16)]
        t2b = thr_v[pl.ds(4 * C + g * 16, 16)]
        t2c = thr_v[pl.ds(5 * C + g * 16, 16)]
        t2d = thr_v[pl.ds(6 * C + g * 16, 16)]
        g3base = iota + (7 * C + g * 16)

        def one_token(t):
            cb = t * (4 * C) + g * 16
            v0 = ch_v[pl.ds(cb, 16)]
            v1 = ch_v[pl.ds(cb + C, 16)]
            v2 = ch_v[pl.ds(cb + 2 * C, 16)]
            v3 = ch_v[pl.ds(cb + 3 * C, 16)]
            b0 = v0 > t0
            b1 = v1 > jnp.where(b0, t1b, t1a)
            td2 = jnp.where(b0, jnp.where(b1, t2d, t2c),
                            jnp.where(b1, t2b, t2a))
            b2 = v2 > td2
            off = (jnp.where(b0, 4, zero16) + jnp.where(b1, 2, zero16)
                   + jnp.where(b2, ones16, zero16))
            t3 = plsc.load_gather(thr_v, [g3base + off * C])
            leaf = 2 * off + jnp.where(v3 > t3, ones16, zero16)
            idx_v[pl.ds(t * C + g * 16, 16)] = leaf.astype(jnp.float32)

        plsc.parallel_loop(0, tpw, unroll=8)(one_token)

    pltpu.sync_copy(idx_v, idx_hbm.at[pl.ds(wid * tpw * C, tpw * C)])


def _sc_encode(x, dims_r, thr_r, tok_off=0, n_tokens=None):
    total_tokens, in_features = x.shape
    if n_tokens is None:
        n_tokens = total_tokens
    tpw = n_tokens // NUM_WORKERS
    mesh = plsc.VectorSubcoreMesh(core_axis_name="c", subcore_axis_name="s",
                                  num_cores=NUM_CORES,
                                  num_subcores=NUM_SUBCORES)
    body = functools.partial(_sc_encode_body_nc, NUM_CORES, n_tokens,
                             in_features, tok_off)
    fn = pl.kernel(
        body,
        out_type=jax.ShapeDtypeStruct((n_tokens * C,), jnp.float32),
        mesh=mesh,
        scratch_types=[
            pltpu.VMEM((4 * C,), jnp.int32),
            pltpu.VMEM((15 * C,), jnp.float32),
            pltpu.VMEM((2 * _CHUNK, in_features), jnp.float32),
            pltpu.VMEM((tpw * 4 * C,), jnp.float32),
            pltpu.VMEM((tpw * C,), jnp.float32),
            pltpu.SemaphoreType.DMA,
        ],
        compiler_params=pltpu.CompilerParams(needs_layout_passes=False),
    )
    return fn(x, dims_r, thr_r)


def _tc_decode_body(idx_ref, r_ref, l_ref, o_ref):
    bm = idx_ref.shape[0]
    idxb = idx_ref[...].astype(jnp.bfloat16)
    rep = lax.dot_general(idxb, r_ref[...], (((1,), (0,)), ((), ())),
                          preferred_element_type=jnp.float32)
    kv = lax.broadcasted_iota(jnp.int32, (bm, C * K), 1) & (K - 1)
    e = (rep == kv.astype(jnp.float32)).astype(jnp.bfloat16)
    o_ref[...] = lax.dot_general(e, l_ref[...], (((1,), (1,)), ((), ())),
                                 preferred_element_type=jnp.float32)


def _tc_decode(idxm, rep_mat, lut_flat):
    n_tokens = idxm.shape[0]
    out_features = lut_flat.shape[0]
    bm = 512
    grid = (n_tokens // bm,)
    return pl.pallas_call(
        _tc_decode_body,
        grid=grid,
        in_specs=[
            pl.BlockSpec((bm, C), lambda i: (i, 0)),
            pl.BlockSpec((C, C * K), lambda i: (0, 0)),
            pl.BlockSpec((out_features, C * K), lambda i: (0, 0)),
        ],
        out_specs=pl.BlockSpec((bm, out_features), lambda i: (i, 0)),
        out_shape=jax.ShapeDtypeStruct((n_tokens, out_features), jnp.float32),
    )(idxm, rep_mat, lut_flat)


def kernel(inputMatrix, dims, thresholds, lut, selectionMatrix, treeDesMat):
    n_tokens, in_features = inputMatrix.shape
    out_features = lut.shape[0]

    dims_r = dims.reshape(C, 4).T.reshape(-1)
    thr_r = _adjust_thresholds(thresholds.reshape(-1)).reshape(C, 15).T.reshape(-1)

    rep_mat = (lax.broadcasted_iota(jnp.int32, (C, C * K), 1) // K ==
               lax.broadcasted_iota(jnp.int32, (C, C * K), 0)
               ).astype(jnp.bfloat16)
    lut_flat = lut.reshape(out_features, C * K).astype(jnp.bfloat16)

    idx_flat = _sc_encode(inputMatrix, dims_r, thr_r)
    idxm = idx_flat.reshape(n_tokens, C)
    return _tc_decode(idxm, rep_mat, lut_flat)
